# baseline (speedup 1.0000x reference)
import functools

import jax
import jax.numpy as jnp
from jax import lax
from jax.experimental import pallas as pl
from jax.experimental.pallas import tpu as pltpu
from jax.experimental.pallas import tpu_sc as plsc

_N = 10000
_E = 320000
_D = 128
_L = 5
_G = 64

_NC = 2
_NS = 16
_NW = _NC * _NS
_EPW = _E // _NW
_K = 80
_NCH = _EPW // _K
_NST = 5
_SB = _NCH // _NST
_RPS = 624


def _sc_body(aug_hbm, idx_hbm, dst_hbm, out_hbm,
             idxs_v, dsts_v, rows0, rows1, agg,
             semg0, semg1, sems0, sems1):
    cid = lax.axis_index("c")
    sid = lax.axis_index("s")
    wid = sid * _NC + cid
    ebase = wid * _EPW

    for i in range(_K):
        for j in range(_D // 16):
            rows0[i, pl.ds(16 * j, 16)] = jnp.zeros((16,), jnp.float32)
    rbase = sid * _RPS
    for cpy in range(_RPS // _K):
        pltpu.sync_copy(rows0, agg.at[pl.ds(rbase + cpy * _K, _K)])
    pltpu.sync_copy(rows0.at[pl.ds(0, _RPS % _K)],
                    agg.at[pl.ds(rbase + _RPS - _RPS % _K, _RPS % _K)])

    @pl.when(sid == _NS - 1)
    def _zero_tail():
        pltpu.sync_copy(rows0.at[pl.ds(0, _N - _NS * _RPS)],
                        agg.at[pl.ds(_NS * _RPS, _N - _NS * _RPS)])

    plsc.subcore_barrier()

    def gather(c, rows, sem):
        pltpu.async_copy(aug_hbm.at[idxs_v.at[pl.ds(c * _K, _K)]], rows, sem)

    def gather_wait(c, rows, sem):
        pltpu.make_async_copy(aug_hbm.at[idxs_v.at[pl.ds(c * _K, _K)]],
                              rows, sem).wait()

    def scatter(c, rows, sem):
        return pltpu.async_copy(rows, agg.at[dsts_v.at[c]], sem, add=True)

    for s in range(_NST):
        pltpu.sync_copy(idx_hbm.at[pl.ds(ebase + s * _SB * _K, _SB * _K)],
                        idxs_v)
        pltpu.sync_copy(dst_hbm.at[wid * _NST + s], dsts_v)
        gather(0, rows0, semg0)
        gather(1, rows1, semg1)

        def pair(i, carry):
            c0 = 2 * i
            gather_wait(c0, rows0, semg0)
            s0 = scatter(c0, rows0, sems0)
            gather_wait(c0 + 1, rows1, semg1)
            s1 = scatter(c0 + 1, rows1, sems1)
            s0.wait()
            gather(c0 + 2, rows0, semg0)
            s1.wait()
            gather(c0 + 3, rows1, semg1)
            return carry

        lax.fori_loop(0, (_SB - 3) // 2, pair, 0)

        cz = _SB - 3
        gather_wait(cz, rows0, semg0)
        s0 = scatter(cz, rows0, sems0)
        gather_wait(cz + 1, rows1, semg1)
        s1 = scatter(cz + 1, rows1, sems1)
        s0.wait()
        gather(cz + 2, rows0, semg0)
        s1.wait()
        gather_wait(cz + 2, rows0, semg0)
        scatter(cz + 2, rows0, sems0).wait()

    plsc.subcore_barrier()
    pltpu.sync_copy(agg.at[pl.ds(sid * _RPS, _RPS)],
                    out_hbm.at[pl.ds(cid * _N + sid * _RPS, _RPS)])

    @pl.when(sid == _NS - 1)
    def _write_tail():
        pltpu.sync_copy(agg.at[pl.ds(_NS * _RPS, _N - _NS * _RPS)],
                        out_hbm.at[pl.ds(cid * _N + _NS * _RPS,
                                         _N - _NS * _RPS)])


@functools.cache
def _sc_segment_sum_kernel():
    return functools.partial(
        pl.kernel,
        out_type=jax.ShapeDtypeStruct((_NC * _N, _D), jnp.float32),
        mesh=plsc.VectorSubcoreMesh(core_axis_name="c", subcore_axis_name="s"),
        scratch_types=[
            pltpu.VMEM((_SB * _K,), jnp.int32),
            pltpu.VMEM((_SB, _K), jnp.int32),
            pltpu.VMEM((_K, _D), jnp.float32),
            pltpu.VMEM((_K, _D), jnp.float32),
            pltpu.VMEM_SHARED((_N, _D), jnp.float32),
            pltpu.SemaphoreType.DMA,
            pltpu.SemaphoreType.DMA,
            pltpu.SemaphoreType.DMA,
            pltpu.SemaphoreType.DMA,
        ],
    )(_sc_body)


def _sc_segment_sum(aug2d, idx3, dst3):
    return _sc_segment_sum_kernel()(aug2d, idx3, dst3)


def _bn(z, g, b):
    mu = jnp.mean(z, axis=0, keepdims=True)
    xc = z - mu
    var = jnp.mean(xc * xc, axis=0, keepdims=True)
    return g * (xc / jnp.sqrt(var + 1e-5)) + b


def _aug_out(h, t_ref, aug_ref):
    for t in range(4):
        aug_ref[t] = jnp.maximum(h + t_ref[t:t + 1, :], 0.0)


def _prep_body(x_ref, t_ref, aug_ref):
    _aug_out(x_ref[...], t_ref, aug_ref)


_prep = pl.pallas_call(
    _prep_body,
    out_shape=jax.ShapeDtypeStruct((4, _N, _D), jnp.float32),
)


def _mlp_core(h_ref, p_ref, eps_ref, w1_ref, b1_ref, g1_ref, be1_ref,
              w2_ref, b2_ref, bng_ref, bnb_ref):
    h = h_ref[...]
    z = (1.0 + eps_ref[0, 0]) * h + (p_ref[0:_N] + p_ref[_N:2 * _N])
    z = jnp.dot(z, w1_ref[...], preferred_element_type=jnp.float32) + b1_ref[...]
    z = jnp.maximum(_bn(z, g1_ref[...], be1_ref[...]), 0.0)
    z = jnp.dot(z, w2_ref[...], preferred_element_type=jnp.float32) + b2_ref[...]
    return jnp.maximum(_bn(z, bng_ref[...], bnb_ref[...]), 0.0)


def _mlp_body(h_ref, p_ref, eps_ref, w1_ref, b1_ref, g1_ref, be1_ref,
              w2_ref, b2_ref, bng_ref, bnb_ref, tn_ref, hn_ref, aug_ref):
    hn = _mlp_core(h_ref, p_ref, eps_ref, w1_ref, b1_ref, g1_ref, be1_ref,
                   w2_ref, b2_ref, bng_ref, bnb_ref)
    hn_ref[...] = hn
    _aug_out(hn, tn_ref, aug_ref)


_mlp = pl.pallas_call(
    _mlp_body,
    out_shape=(jax.ShapeDtypeStruct((_N, _D), jnp.float32),
               jax.ShapeDtypeStruct((4, _N, _D), jnp.float32)),
)


def _final_body(h_ref, p_ref, eps_ref, w1_ref, b1_ref, g1_ref, be1_ref,
                w2_ref, b2_ref, bng_ref, bnb_ref, sid_ref, out_ref):
    hn = _mlp_core(h_ref, p_ref, eps_ref, w1_ref, b1_ref, g1_ref, be1_ref,
                   w2_ref, b2_ref, bng_ref, bnb_ref)
    seg = lax.broadcasted_iota(jnp.int32, (_N, _G), 1)
    onehot = (sid_ref[...] == seg).astype(jnp.float32)
    sums = lax.dot_general(onehot, hn, (((0,), (0,)), ((), ())),
                           preferred_element_type=jnp.float32)
    counts = jnp.sum(onehot, axis=0)
    out_ref[...] = sums / jnp.maximum(counts, 1.0)[:, None]


_final = pl.pallas_call(
    _final_body,
    out_shape=jax.ShapeDtypeStruct((_G, _D), jnp.float32),
)


def kernel(x, edge_index, edge_attr, batch, num_subgraphs, subgraph_batch,
           bond_tables, W1, b1, g1, be1, W2, b2, eps, bn_g, bn_b):
    src = edge_index[0]
    dst = edge_index[1]
    ea = edge_attr.reshape(-1).astype(jnp.int32)
    gidx = ea * _N + src

    offsets = jnp.concatenate(
        [jnp.zeros((1,), num_subgraphs.dtype), jnp.cumsum(num_subgraphs)])
    sub_id = (subgraph_batch + jnp.take(offsets, batch)).astype(jnp.int32)
    sub_id = sub_id.reshape(_N, 1)

    def args(l):
        return (eps[l].reshape(1, 1), W1[l], b1[l].reshape(1, _D),
                g1[l].reshape(1, _D), be1[l].reshape(1, _D), W2[l],
                b2[l].reshape(1, _D), bn_g[l].reshape(1, _D),
                bn_b[l].reshape(1, _D))

    h = x
    aug = _prep(x, bond_tables[0])
    for l in range(_L):
        part = _sc_segment_sum(aug.reshape(4 * _N, _D), gidx,
                               dst.reshape(_NW * _NST, _SB, _K))
        if l + 1 < _L:
            h, aug = _mlp(h, part, *args(l), bond_tables[l + 1])
        else:
            out = _final(h, part, *args(l), sub_id)
    return out


# speedup vs baseline: 1.3568x; 1.3568x over previous
import functools

import jax
import jax.numpy as jnp
from jax import lax
from jax.experimental import pallas as pl
from jax.experimental.pallas import tpu as pltpu
from jax.experimental.pallas import tpu_sc as plsc

_N = 10000
_E = 320000
_D = 128
_L = 5
_G = 64

_NC = 2
_NS = 16
_NW = _NC * _NS
_EPW = _E // _NW
_K = 80
_NCH = _EPW // _K
_NST = 5
_SB = _NCH // _NST
_RPS = 624


def _sc_body(aug_hbm, idx_hbm, dst_hbm, out_hbm,
             idxs_v, dsts_v, rows0, rows1, rows2, agg,
             semg0, semg1, semg2, sems0, sems1, sems2):
    cid = lax.axis_index("c")
    sid = lax.axis_index("s")
    wid = sid * _NC + cid
    ebase = wid * _EPW
    semg = (semg0, semg1, semg2)
    sems = (sems0, sems1, sems2)

    for i in range(_K):
        for j in range(_D // 16):
            rows0[i, pl.ds(16 * j, 16)] = jnp.zeros((16,), jnp.float32)
    rbase = sid * _RPS
    for cpy in range(_RPS // _K):
        pltpu.sync_copy(rows0, agg.at[pl.ds(rbase + cpy * _K, _K)])
    pltpu.sync_copy(rows0.at[pl.ds(0, _RPS % _K)],
                    agg.at[pl.ds(rbase + _RPS - _RPS % _K, _RPS % _K)])

    @pl.when(sid == _NS - 1)
    def _zero_tail():
        pltpu.sync_copy(rows0.at[pl.ds(0, _N - _NS * _RPS)],
                        agg.at[pl.ds(_NS * _RPS, _N - _NS * _RPS)])

    plsc.subcore_barrier()

    rows = (rows0, rows1, rows2)

    def gather(c, b):
        pltpu.async_copy(aug_hbm.at[idxs_v.at[pl.ds(c * _K, _K)]],
                         rows[b], semg[b])

    def gather_wait(b):
        pltpu.make_async_copy(aug_hbm.at[pl.ds(0, _K)], rows[b],
                              semg[b]).wait()

    def scatter(c, b):
        pltpu.async_copy(rows[b], agg.at[dsts_v.at[c]], sems[b], add=True)

    def scatter_wait(b):
        pltpu.make_async_copy(aug_hbm.at[pl.ds(0, _K)], rows[b],
                              sems[b]).wait()

    for s in range(_NST):
        pltpu.sync_copy(idx_hbm.at[pl.ds(ebase + s * _SB * _K, _SB * _K)],
                        idxs_v)
        pltpu.sync_copy(dst_hbm.at[wid * _NST + s], dsts_v)
        gather(0, 0)
        gather(1, 1)
        gather_wait(0)
        scatter(0, 0)
        gather(2, 2)

        def tri(i, carry):
            c0 = 3 * i + 1
            gather_wait(1)
            scatter(c0, 1)
            scatter_wait(0)
            gather(c0 + 2, 0)
            gather_wait(2)
            scatter(c0 + 1, 2)
            scatter_wait(1)
            gather(c0 + 3, 1)
            gather_wait(0)
            scatter(c0 + 2, 0)
            scatter_wait(2)
            gather(c0 + 4, 2)
            return carry

        lax.fori_loop(0, (_SB - 4) // 3, tri, 0)

        cz = _SB - 3
        gather_wait(1)
        scatter(cz, 1)
        scatter_wait(0)
        gather(cz + 2, 0)
        gather_wait(2)
        scatter(cz + 1, 2)
        scatter_wait(1)
        gather_wait(0)
        scatter(cz + 2, 0)
        scatter_wait(2)
        scatter_wait(0)

    plsc.subcore_barrier()
    pltpu.sync_copy(agg.at[pl.ds(sid * _RPS, _RPS)],
                    out_hbm.at[pl.ds(cid * _N + sid * _RPS, _RPS)])

    @pl.when(sid == _NS - 1)
    def _write_tail():
        pltpu.sync_copy(agg.at[pl.ds(_NS * _RPS, _N - _NS * _RPS)],
                        out_hbm.at[pl.ds(cid * _N + _NS * _RPS,
                                         _N - _NS * _RPS)])


@functools.cache
def _sc_segment_sum_kernel():
    return functools.partial(
        pl.kernel,
        out_type=jax.ShapeDtypeStruct((_NC * _N, _D), jnp.float32),
        mesh=plsc.VectorSubcoreMesh(core_axis_name="c", subcore_axis_name="s"),
        scratch_types=[
            pltpu.VMEM((_SB * _K,), jnp.int32),
            pltpu.VMEM((_SB, _K), jnp.int32),
            pltpu.VMEM((_K, _D), jnp.float32),
            pltpu.VMEM((_K, _D), jnp.float32),
            pltpu.VMEM((_K, _D), jnp.float32),
            pltpu.VMEM_SHARED((_N, _D), jnp.float32),
            pltpu.SemaphoreType.DMA,
            pltpu.SemaphoreType.DMA,
            pltpu.SemaphoreType.DMA,
            pltpu.SemaphoreType.DMA,
            pltpu.SemaphoreType.DMA,
            pltpu.SemaphoreType.DMA,
        ],
    )(_sc_body)


def _sc_segment_sum(aug2d, idx3, dst3):
    return _sc_segment_sum_kernel()(aug2d, idx3, dst3)


def _bn(z, g, b):
    mu = jnp.mean(z, axis=0, keepdims=True)
    xc = z - mu
    var = jnp.mean(xc * xc, axis=0, keepdims=True)
    return g * (xc / jnp.sqrt(var + 1e-5)) + b


def _aug_out(h, t_ref, aug_ref):
    for t in range(4):
        aug_ref[t] = jnp.maximum(h + t_ref[t:t + 1, :], 0.0)


def _prep_body(x_ref, t_ref, aug_ref):
    _aug_out(x_ref[...], t_ref, aug_ref)


_prep = pl.pallas_call(
    _prep_body,
    out_shape=jax.ShapeDtypeStruct((4, _N, _D), jnp.float32),
)


def _mlp_core(h_ref, p_ref, eps_ref, w1_ref, b1_ref, g1_ref, be1_ref,
              w2_ref, b2_ref, bng_ref, bnb_ref):
    h = h_ref[...]
    z = (1.0 + eps_ref[0, 0]) * h + (p_ref[0:_N] + p_ref[_N:2 * _N])
    z = jnp.dot(z, w1_ref[...], preferred_element_type=jnp.float32) + b1_ref[...]
    z = jnp.maximum(_bn(z, g1_ref[...], be1_ref[...]), 0.0)
    z = jnp.dot(z, w2_ref[...], preferred_element_type=jnp.float32) + b2_ref[...]
    return jnp.maximum(_bn(z, bng_ref[...], bnb_ref[...]), 0.0)


def _mlp_body(h_ref, p_ref, eps_ref, w1_ref, b1_ref, g1_ref, be1_ref,
              w2_ref, b2_ref, bng_ref, bnb_ref, tn_ref, hn_ref, aug_ref):
    hn = _mlp_core(h_ref, p_ref, eps_ref, w1_ref, b1_ref, g1_ref, be1_ref,
                   w2_ref, b2_ref, bng_ref, bnb_ref)
    hn_ref[...] = hn
    _aug_out(hn, tn_ref, aug_ref)


_mlp = pl.pallas_call(
    _mlp_body,
    out_shape=(jax.ShapeDtypeStruct((_N, _D), jnp.float32),
               jax.ShapeDtypeStruct((4, _N, _D), jnp.float32)),
)


def _final_body(h_ref, p_ref, eps_ref, w1_ref, b1_ref, g1_ref, be1_ref,
                w2_ref, b2_ref, bng_ref, bnb_ref, sid_ref, out_ref):
    hn = _mlp_core(h_ref, p_ref, eps_ref, w1_ref, b1_ref, g1_ref, be1_ref,
                   w2_ref, b2_ref, bng_ref, bnb_ref)
    seg = lax.broadcasted_iota(jnp.int32, (_N, _G), 1)
    onehot = (sid_ref[...] == seg).astype(jnp.float32)
    sums = lax.dot_general(onehot, hn, (((0,), (0,)), ((), ())),
                           preferred_element_type=jnp.float32)
    counts = jnp.sum(onehot, axis=0)
    out_ref[...] = sums / jnp.maximum(counts, 1.0)[:, None]


_final = pl.pallas_call(
    _final_body,
    out_shape=jax.ShapeDtypeStruct((_G, _D), jnp.float32),
)


def kernel(x, edge_index, edge_attr, batch, num_subgraphs, subgraph_batch,
           bond_tables, W1, b1, g1, be1, W2, b2, eps, bn_g, bn_b):
    src = edge_index[0]
    dst = edge_index[1]
    ea = edge_attr.reshape(-1).astype(jnp.int32)
    gidx = ea * _N + src

    offsets = jnp.concatenate(
        [jnp.zeros((1,), num_subgraphs.dtype), jnp.cumsum(num_subgraphs)])
    sub_id = (subgraph_batch + jnp.take(offsets, batch)).astype(jnp.int32)
    sub_id = sub_id.reshape(_N, 1)

    def args(l):
        return (eps[l].reshape(1, 1), W1[l], b1[l].reshape(1, _D),
                g1[l].reshape(1, _D), be1[l].reshape(1, _D), W2[l],
                b2[l].reshape(1, _D), bn_g[l].reshape(1, _D),
                bn_b[l].reshape(1, _D))

    h = x
    aug = _prep(x, bond_tables[0])
    for l in range(_L):
        part = _sc_segment_sum(aug.reshape(4 * _N, _D), gidx,
                               dst.reshape(_NW * _NST, _SB, _K))
        if l + 1 < _L:
            h, aug = _mlp(h, part, *args(l), bond_tables[l + 1])
        else:
            out = _final(h, part, *args(l), sub_id)
    return out
